# baseline (speedup 1.0000x reference)
import functools

import jax
import jax.numpy as jnp
from jax import lax
from jax.experimental import pallas as pl
from jax.experimental.pallas import tpu as pltpu
from jax.experimental.pallas import tpu_sc as plsc


def _sc_row_gather(table, idx):
    n, d = table.shape
    info = plsc.get_sparse_core_info()
    nw = info.num_cores * info.num_subcores
    rows_per_w = n // nw
    assert rows_per_w * nw == n
    chunk = min(32, rows_per_w)
    n_chunks = rows_per_w // chunk
    assert n_chunks * chunk == rows_per_w

    mesh = plsc.VectorSubcoreMesh(core_axis_name="c", subcore_axis_name="s")

    @functools.partial(
        pl.kernel,
        out_type=jax.ShapeDtypeStruct((n, d), table.dtype),
        mesh=mesh,
        scratch_types=[
            pltpu.VMEM((chunk,), jnp.int32),
            pltpu.VMEM((chunk, d), table.dtype),
            pltpu.SemaphoreType.DMA,
        ],
    )
    def gather_kernel(table_hbm, idx_hbm, out_hbm, idx_v, rows_v, sem):
        wid = lax.axis_index("s") * info.num_cores + lax.axis_index("c")
        base = wid * rows_per_w

        def body(c, carry):
            row0 = base + c * chunk
            pltpu.sync_copy(idx_hbm.at[pl.ds(row0, chunk)], idx_v)
            pltpu.async_copy(table_hbm.at[idx_v], rows_v, sem).wait()
            pltpu.sync_copy(rows_v, out_hbm.at[pl.ds(row0, chunk), :])
            return carry

        lax.fori_loop(0, n_chunks, body, 0)

    return gather_kernel(table, idx)


def _ffn_body(g_ref, m_ref, rs_ref, re_ref,
              x_ref, w1_ref, b1_ref, w2_ref, b2_ref, o_ref,
              *, tm, kf_total):
    t = pl.program_id(0)
    kf = pl.program_id(1)

    prev_m = m_ref[jnp.maximum(t - 1, 0)]
    is_new_block = jnp.logical_or(t == 0, m_ref[t] != prev_m)

    @pl.when(jnp.logical_and(kf == 0, is_new_block))
    def _init():
        o_ref[...] = jnp.zeros_like(o_ref)

    rows = m_ref[t] * tm + lax.broadcasted_iota(jnp.int32, (tm, 1), 0)
    active = jnp.logical_and(rows >= rs_ref[t], rows < re_ref[t])

    x = x_ref[...].astype(jnp.bfloat16)
    h = jnp.dot(x, w1_ref[0].astype(jnp.bfloat16),
                preferred_element_type=jnp.float32)
    h = jax.nn.gelu(h + b1_ref[0, 0][None, :])
    h = jnp.where(active, h, 0.0).astype(jnp.bfloat16)
    o_ref[...] += jnp.dot(h, w2_ref[0].astype(jnp.bfloat16),
                          preferred_element_type=jnp.float32)

    @pl.when(kf == kf_total - 1)
    def _bias():
        o_ref[...] += jnp.where(active, b2_ref[0, 0][None, :], 0.0)


def _grouped_ffn(x_sorted, w1, b1, w2, b2, g_ids, m_ids, rs, re, tm, tf,
                 interpret=False):
    n, d = x_sorted.shape
    e, _, ff = w1.shape
    t_slots = g_ids.shape[0]
    kf_total = ff // tf

    grid_spec = pltpu.PrefetchScalarGridSpec(
        num_scalar_prefetch=4,
        grid=(t_slots, kf_total),
        in_specs=[
            pl.BlockSpec((tm, d), lambda t, kf, g, m, rs, re: (m[t], 0)),
            pl.BlockSpec((1, d, tf), lambda t, kf, g, m, rs, re: (g[t], 0, kf)),
            pl.BlockSpec((1, 1, tf), lambda t, kf, g, m, rs, re: (g[t], 0, kf)),
            pl.BlockSpec((1, tf, d), lambda t, kf, g, m, rs, re: (g[t], kf, 0)),
            pl.BlockSpec((1, 1, d), lambda t, kf, g, m, rs, re: (g[t], 0, 0)),
        ],
        out_specs=pl.BlockSpec((tm, d), lambda t, kf, g, m, rs, re: (m[t], 0)),
    )
    return pl.pallas_call(
        functools.partial(_ffn_body, tm=tm, kf_total=kf_total),
        grid_spec=grid_spec,
        out_shape=jax.ShapeDtypeStruct((n, d), x_sorted.dtype),
        compiler_params=pltpu.CompilerParams(
            dimension_semantics=("arbitrary", "arbitrary"),
        ),
        interpret=interpret,
    )(g_ids, m_ids, rs, re, x_sorted, w1,
      b1.reshape(e, 1, ff), w2, b2.reshape(e, 1, d))


def _logical_tiles(offsets, n, e, tm):
    tiles_m = n // tm
    t_slots = tiles_m + e - 1
    counts = offsets[1:] - offsets[:-1]
    first = offsets[:-1] // tm
    last = jnp.where(counts > 0, (offsets[1:] - 1) // tm, first)
    ntiles = jnp.where(counts > 0, last - first + 1, 1)
    base = jnp.concatenate([jnp.zeros((1,), jnp.int32),
                            jnp.cumsum(ntiles).astype(jnp.int32)])

    tt = jnp.arange(t_slots, dtype=jnp.int32)
    g = jnp.searchsorted(base, tt, side="right").astype(jnp.int32) - 1
    g_c = jnp.clip(g, 0, e - 1)
    valid = tt < base[-1]
    j = tt - base[g_c]
    m_raw = first[g_c] + j
    rs = jnp.where(valid, jnp.maximum(offsets[g_c], m_raw * tm), 0)
    re = jnp.where(valid, jnp.minimum(offsets[g_c + 1], (m_raw + 1) * tm), 0)
    m_ids = jnp.clip(m_raw, 0, tiles_m - 1).astype(jnp.int32)
    return g_c, m_ids, rs.astype(jnp.int32), re.astype(jnp.int32)


def kernel(hidden_states, token_type_ids, W1, b1, W2, b2):
    b, s, d = hidden_states.shape
    e, _, ff = W1.shape
    n = b * s
    tm, tf = 1024, 512

    flat = hidden_states.reshape(n, d)
    tt = token_type_ids.reshape(n).astype(jnp.int32)

    perm = jnp.argsort(tt).astype(jnp.int32)
    inv_perm = jnp.zeros((n,), jnp.int32).at[perm].set(
        jnp.arange(n, dtype=jnp.int32))
    counts = jnp.sum(tt[None, :] == jnp.arange(e, dtype=jnp.int32)[:, None],
                     axis=1, dtype=jnp.int32)
    offsets = jnp.concatenate([jnp.zeros((1,), jnp.int32),
                               jnp.cumsum(counts).astype(jnp.int32)])
    g_ids, m_ids, rs, re = _logical_tiles(offsets, n, e, tm)

    x_sorted = _sc_row_gather(flat, perm)
    y_sorted = _grouped_ffn(x_sorted, W1, b1, W2, b2,
                            g_ids, m_ids, rs, re, tm, tf)
    out = _sc_row_gather(y_sorted, inv_perm)
    return out.reshape(b, s, d)


# speedup vs baseline: 1.1882x; 1.1882x over previous
import functools

import jax
import jax.numpy as jnp
from jax import lax
from jax.experimental import pallas as pl
from jax.experimental.pallas import tpu as pltpu
from jax.experimental.pallas import tpu_sc as plsc


def _sc_row_gather(table, idx):
    n, d = table.shape
    info = plsc.get_sparse_core_info()
    nw = info.num_cores * info.num_subcores
    rows_per_w = n // nw
    assert rows_per_w * nw == n
    chunk = min(32, rows_per_w)
    n_chunks = rows_per_w // chunk
    assert n_chunks * chunk == rows_per_w

    mesh = plsc.VectorSubcoreMesh(core_axis_name="c", subcore_axis_name="s")

    @functools.partial(
        pl.kernel,
        out_type=jax.ShapeDtypeStruct((n, d), table.dtype),
        mesh=mesh,
        scratch_types=[
            pltpu.VMEM((chunk,), jnp.int32),
            pltpu.VMEM((chunk, d), table.dtype),
            pltpu.SemaphoreType.DMA,
        ],
    )
    def gather_kernel(table_hbm, idx_hbm, out_hbm, idx_v, rows_v, sem):
        wid = lax.axis_index("s") * info.num_cores + lax.axis_index("c")
        base = wid * rows_per_w

        def body(c, carry):
            row0 = base + c * chunk
            pltpu.sync_copy(idx_hbm.at[pl.ds(row0, chunk)], idx_v)
            pltpu.async_copy(table_hbm.at[idx_v], rows_v, sem).wait()
            pltpu.sync_copy(rows_v, out_hbm.at[pl.ds(row0, chunk), :])
            return carry

        lax.fori_loop(0, n_chunks, body, 0)

    return gather_kernel(table, idx)


def _ffn_body(g_ref, m_ref, rs_ref, re_ref,
              x_ref, w1_ref, b1_ref, w2_ref, b2_ref, o_ref,
              *, tm, kf_total):
    t = pl.program_id(0)
    kf = pl.program_id(1)

    prev_m = m_ref[jnp.maximum(t - 1, 0)]
    is_new_block = jnp.logical_or(t == 0, m_ref[t] != prev_m)
    overwrite = jnp.logical_and(kf == 0, is_new_block)

    rows = m_ref[t] * tm + lax.broadcasted_iota(jnp.int32, (tm, 1), 0)
    active = jnp.logical_and(rows >= rs_ref[t], rows < re_ref[t])

    x = x_ref[...]
    h = jnp.dot(x, w1_ref[0], preferred_element_type=jnp.float32)
    h = jax.nn.gelu(h + b1_ref[0, 0][None, :])
    h = jnp.where(active, h, 0.0)
    acc = jnp.dot(h, w2_ref[0], preferred_element_type=jnp.float32)
    bias_mask = jnp.logical_and(active, kf == kf_total - 1)
    acc = acc + jnp.where(bias_mask, b2_ref[0, 0][None, :], 0.0)

    @pl.when(overwrite)
    def _store():
        o_ref[...] = acc

    @pl.when(jnp.logical_not(overwrite))
    def _accum():
        o_ref[...] += acc


def _grouped_ffn(x_sorted, w1, b1, w2, b2, g_ids, m_ids, rs, re, tm, tf,
                 interpret=False):
    n, d = x_sorted.shape
    e, _, ff = w1.shape
    t_slots = g_ids.shape[0]
    kf_total = ff // tf

    grid_spec = pltpu.PrefetchScalarGridSpec(
        num_scalar_prefetch=4,
        grid=(t_slots, kf_total),
        in_specs=[
            pl.BlockSpec((tm, d), lambda t, kf, g, m, rs, re: (m[t], 0)),
            pl.BlockSpec((1, d, tf), lambda t, kf, g, m, rs, re: (g[t], 0, kf)),
            pl.BlockSpec((1, 1, tf), lambda t, kf, g, m, rs, re: (g[t], 0, kf)),
            pl.BlockSpec((1, tf, d), lambda t, kf, g, m, rs, re: (g[t], kf, 0)),
            pl.BlockSpec((1, 1, d), lambda t, kf, g, m, rs, re: (g[t], 0, 0)),
        ],
        out_specs=pl.BlockSpec((tm, d), lambda t, kf, g, m, rs, re: (m[t], 0)),
    )
    return pl.pallas_call(
        functools.partial(_ffn_body, tm=tm, kf_total=kf_total),
        grid_spec=grid_spec,
        out_shape=jax.ShapeDtypeStruct((n, d), x_sorted.dtype),
        compiler_params=pltpu.CompilerParams(
            dimension_semantics=("arbitrary", "arbitrary"),
        ),
        interpret=interpret,
    )(g_ids, m_ids, rs, re, x_sorted, w1,
      b1.reshape(e, 1, ff), w2, b2.reshape(e, 1, d))


def _logical_tiles(offsets, n, e, tm):
    tiles_m = n // tm
    t_slots = tiles_m + e - 1
    counts = offsets[1:] - offsets[:-1]
    first = offsets[:-1] // tm
    last = jnp.where(counts > 0, (offsets[1:] - 1) // tm, first)
    ntiles = jnp.where(counts > 0, last - first + 1, 1)
    base = jnp.concatenate([jnp.zeros((1,), jnp.int32),
                            jnp.cumsum(ntiles).astype(jnp.int32)])

    tt = jnp.arange(t_slots, dtype=jnp.int32)
    g = jnp.searchsorted(base, tt, side="right").astype(jnp.int32) - 1
    g_c = jnp.clip(g, 0, e - 1)
    valid = tt < base[-1]
    j = tt - base[g_c]
    m_raw = first[g_c] + j
    rs = jnp.where(valid, jnp.maximum(offsets[g_c], m_raw * tm), 0)
    re = jnp.where(valid, jnp.minimum(offsets[g_c + 1], (m_raw + 1) * tm), 0)
    m_ids = jnp.clip(m_raw, 0, tiles_m - 1).astype(jnp.int32)
    return g_c, m_ids, rs.astype(jnp.int32), re.astype(jnp.int32)


def kernel(hidden_states, token_type_ids, W1, b1, W2, b2):
    b, s, d = hidden_states.shape
    e, _, ff = W1.shape
    n = b * s
    tm, tf = 512, 1024

    flat = hidden_states.reshape(n, d)
    tt = token_type_ids.reshape(n).astype(jnp.int32)

    perm = jnp.argsort(tt).astype(jnp.int32)
    inv_perm = jnp.zeros((n,), jnp.int32).at[perm].set(
        jnp.arange(n, dtype=jnp.int32))
    counts = jnp.sum(tt[None, :] == jnp.arange(e, dtype=jnp.int32)[:, None],
                     axis=1, dtype=jnp.int32)
    offsets = jnp.concatenate([jnp.zeros((1,), jnp.int32),
                               jnp.cumsum(counts).astype(jnp.int32)])
    g_ids, m_ids, rs, re = _logical_tiles(offsets, n, e, tm)

    x_sorted = _sc_row_gather(flat, perm)
    y_sorted = _grouped_ffn(x_sorted, W1, b1, W2, b2,
                            g_ids, m_ids, rs, re, tm, tf)
    out = _sc_row_gather(y_sorted, inv_perm)
    return out.reshape(b, s, d)
